# Initial kernel scaffold; baseline (speedup 1.0000x reference)
import jax
import jax.numpy as jnp
from jax.experimental import pallas as pl


def kernel(kp1, kp2, weights):
    raise NotImplementedError("write your pallas kernel here")


# speedup vs baseline: 12.0582x; 12.0582x over previous
import jax
import jax.numpy as jnp
from jax import lax
from jax.experimental import pallas as pl
from jax.experimental.pallas import tpu as pltpu

N = 10000
NP = 10240
PR, PC = 8, 1280
B = 1024
HB = 8
INL_TH = 2.0
SQRT2 = 1.4142135623730951


def _bf(v):
    return lax.convert_element_type(
        lax.convert_element_type(v, jnp.bfloat16), jnp.float32)


def _rotl(x, r):
    return lax.shift_left(x, jnp.int32(r)) | lax.shift_right_logical(
        x, jnp.int32(32 - r))


def _threefry_bits(ks0, ks1, x1):
    ks2 = ks0 ^ ks1 ^ jnp.int32(0x1BD11BDA)
    ks = [ks0, ks1, ks2]
    rots = [[13, 15, 26, 6], [17, 29, 16, 24]]
    x0 = jnp.zeros_like(x1) + ks0
    x1 = x1 + ks1
    for i in range(5):
        for r in rots[i % 2]:
            x0 = x0 + x1
            x1 = _rotl(x1, r)
            x1 = x0 ^ x1
        x0 = x0 + ks[(i + 1) % 3]
        x1 = x1 + ks[(i + 2) % 3] + jnp.int32(i + 1)
    return x0 ^ x1


def _sample_kernel(key_ref, kps_ref, coords_ref):
    g = pl.program_id(0)
    h0 = g * HB
    r_iota = lax.broadcasted_iota(jnp.int32, (HB, NP), 0)
    c_iota = lax.broadcasted_iota(jnp.int32, (HB, NP), 1)
    L = (h0 + r_iota) * N + c_iota
    bits = _threefry_bits(key_ref[0], key_ref[1], L)
    v = lax.shift_right_logical(bits, jnp.int32(9))
    v = jnp.where(c_iota < N, v, -1)
    coord_rows = [kps_ref[i:i + 1, :] for i in range(4)]
    out = [[], [], [], []]
    for _t in range(4):
        m = jnp.max(v, axis=1, keepdims=True)
        idx = jnp.min(jnp.where(v == m, c_iota, NP), axis=1, keepdims=True)
        sel = c_iota == idx
        for s in range(4):
            out[s].append(
                jnp.sum(jnp.where(sel, coord_rows[s], 0.0), axis=1,
                        keepdims=True))
        v = jnp.where(sel, -1, v)
    coords_ref[...] = jnp.concatenate(out[0] + out[1] + out[2] + out[3],
                                      axis=1)


def _sample(key2, kps_wide):
    return pl.pallas_call(
        _sample_kernel,
        grid=(B // HB,),
        in_specs=[
            pl.BlockSpec(memory_space=pltpu.SMEM),
            pl.BlockSpec((8, NP), lambda g: (0, 0)),
        ],
        out_specs=pl.BlockSpec((HB, 16), lambda g: (g, 0)),
        out_shape=jax.ShapeDtypeStruct((B, 16), jnp.float32),
    )(key2, kps_wide)


def _inv3(T):
    a, b, c = T[0]
    d, e, f = T[1]
    g, h, i = T[2]
    A11 = e * i - f * h
    A12 = c * h - b * i
    A13 = b * f - c * e
    A21 = f * g - d * i
    A22 = a * i - c * g
    A23 = c * d - a * f
    A31 = d * h - e * g
    A32 = b * g - a * h
    A33 = a * e - b * d
    det = a * A11 + b * A21 + c * A31
    return [[A11 / det, A12 / det, A13 / det],
            [A21 / det, A22 / det, A23 / det],
            [A31 / det, A32 / det, A33 / det]]


def _bfv(v):
    if isinstance(v, (int, float)):
        return v
    return _bf(v)


def _mat3bf(Ma, Mb):
    out = []
    for i in range(3):
        row = []
        for j in range(3):
            p = [_bfv(Ma[i][k]) * _bfv(Mb[k][j]) for k in range(3)]
            row.append((p[0] + p[1]) + p[2])
        out.append(row)
    return out


def _solve8(AtA):
    def at(j, k):
        return AtA[(j, k)]

    M = [[at(j, k) + (1e-8 if j == k else 0.0) for k in range(8)]
         + [-at(j, 8)] for j in range(8)]
    for k in range(8):
        piv = M[k][k]
        for r in range(k + 1, 8):
            f = M[r][k] / piv
            for c in range(k + 1, 9):
                M[r][c] = M[r][c] - f * M[k][c]
    xs = [None] * 8
    for k in range(7, -1, -1):
        s = M[k][8]
        for c in range(k + 1, 8):
            s = s - M[k][c] * xs[c]
        xs[k] = s / M[k][k]
    return xs + [None]


def _dlt_core(x1n, y1n, x2n, y2n, T1, T2, accumulate):
    AtA = accumulate(x1n, y1n, x2n, y2n)
    xs = _solve8(AtA)
    one = jnp.ones_like(xs[0])
    Hn = [[xs[0], xs[1], xs[2]], [xs[3], xs[4], xs[5]], [xs[6], xs[7], one]]
    H = _mat3bf(_mat3bf(_inv3(T2), Hn), T1)
    z22 = H[2][2] + 1e-8
    return [[H[i][j] / z22 for j in range(3)] for i in range(3)]


def _ax_ay(X1, Y1, X2, Y2, one):
    ax = [None, None, None, -X1, -Y1, -one, Y2 * X1, Y2 * Y1, Y2]
    ay = [X1, Y1, one, None, None, None, (-X2) * X1, (-X2) * Y1, -X2]
    return ax, ay


def _dlt4_kernel(ct_ref, models_ref, valid_ref):
    x1 = [ct_ref[i] for i in range(4)]
    y1 = [ct_ref[4 + i] for i in range(4)]
    x2 = [ct_ref[8 + i] for i in range(4)]
    y2 = [ct_ref[12 + i] for i in range(4)]

    def norm4(xs, ys):
        mx = (((xs[0] + xs[1]) + xs[2]) + xs[3]) * 0.25
        my = (((ys[0] + ys[1]) + ys[2]) + ys[3]) * 0.25
        d = [jnp.sqrt((xs[i] - mx) ** 2 + (ys[i] - my) ** 2)
             for i in range(4)]
        sc = (((d[0] + d[1]) + d[2]) + d[3]) * 0.25
        sc = SQRT2 / (sc + 1e-8)
        xn = [(xs[i] - mx) * sc for i in range(4)]
        yn = [(ys[i] - my) * sc for i in range(4)]
        T = [[sc, 0.0, -sc * mx], [0.0, sc, -sc * my], [0.0, 0.0, 1.0]]
        return xn, yn, T

    x1n, y1n, T1 = norm4(x1, y1)
    x2n, y2n, T2 = norm4(x2, y2)
    one = jnp.ones_like(x1n[0])

    def accumulate(x1n, y1n, x2n, y2n):
        rows = []
        for i in range(4):
            ax, ay = _ax_ay(x1n[i], y1n[i], x2n[i], y2n[i], one)
            rows.append(ax)
        for i in range(4):
            ax, ay = _ax_ay(x1n[i], y1n[i], x2n[i], y2n[i], one)
            rows.append(ay)
        rows_bf = [[None if e is None else _bf(e) for e in row]
                   for row in rows]
        AtA = {}
        for row in rows_bf:
            for j in range(9):
                if row[j] is None:
                    continue
                for k in range(j, 9):
                    if row[k] is None:
                        continue
                    p = row[j] * row[k]
                    AtA[(j, k)] = AtA.get((j, k), 0.0) + p
        for j in range(9):
            for k in range(j, 9):
                AtA.setdefault((j, k), jnp.zeros_like(one))
                AtA[(k, j)] = AtA[(j, k)]
        return AtA

    H = _dlt_core(x1n, y1n, x2n, y2n, T1, T2, accumulate)

    for i in range(3):
        for j in range(3):
            models_ref[3 * i + j] = H[i][j]

    TRIPLETS = [(0, 1, 2), (0, 1, 3), (0, 2, 3), (1, 2, 3)]

    def det3h(xs, ys, t):
        xa, xb, xc = xs[t[0]], xs[t[1]], xs[t[2]]
        ya, yb, yc = ys[t[0]], ys[t[1]], ys[t[2]]
        return (xa * (yb - yc) - ya * (xb - xc)) + (xb * yc - yb * xc)

    mask = None
    for t in TRIPLETS:
        d1 = det3h(x1, y1, t)
        d2 = det3h(x2, y2, t)
        ok = (d1 * d2) > 0.0
        mask = ok if mask is None else (mask & ok)
    absd = [jnp.abs(H[0][0]), jnp.abs(H[1][1]), jnp.abs(H[2][2])]
    good = jnp.minimum(jnp.minimum(absd[0], absd[1]), absd[2]) > 1e-4
    valid_ref[...] = (mask & good).astype(jnp.float32)


def _dlt4(coordsT):
    return pl.pallas_call(
        _dlt4_kernel,
        out_shape=(jax.ShapeDtypeStruct((9, 8, 128), jnp.float32),
                   jax.ShapeDtypeStruct((8, 128), jnp.float32)),
    )(coordsT)


def _verify_kernel(models_ref, kps_ref, valid_ref, scores_ref):
    mm = models_ref[...]
    h = [mm[:, j:j + 1] for j in range(9)]
    x1 = kps_ref[0:1, :]
    y1 = kps_ref[1:2, :]
    x2 = kps_ref[2:3, :]
    y2 = kps_ref[3:4, :]
    c_iota = lax.broadcasted_iota(jnp.int32, (HB, NP), 1)
    err = _transfer_err(h, x1, y1, x2, y2)
    inl = (err <= INL_TH) & (c_iota < N)
    score = jnp.sum(inl.astype(jnp.float32), axis=1, keepdims=True)
    scores_ref[...] = jnp.where(valid_ref[...] > 0.0, score, -1.0)


def _verify_batch(models_r, kps_wide, valid_r):
    return pl.pallas_call(
        _verify_kernel,
        grid=(B // HB,),
        in_specs=[
            pl.BlockSpec((HB, 9), lambda g: (g, 0)),
            pl.BlockSpec((8, NP), lambda g: (0, 0)),
            pl.BlockSpec((HB, 1), lambda g: (g, 0)),
        ],
        out_specs=pl.BlockSpec((HB, 1), lambda g: (g, 0)),
        out_shape=jax.ShapeDtypeStruct((B, 1), jnp.float32),
    )(models_r, kps_wide, valid_r)


def _argmax_kernel(scores_ref, modelsT_ref, best_ref):
    s = scores_ref[...]
    m = jnp.max(jnp.max(s, axis=1, keepdims=True), axis=0, keepdims=True)
    r_iota = lax.broadcasted_iota(jnp.int32, (8, 128), 0)
    c_iota = lax.broadcasted_iota(jnp.int32, (8, 128), 1)
    flat = r_iota * 128 + c_iota
    eq = s == m
    bidx = jnp.min(jnp.min(jnp.where(eq, flat, B), axis=1, keepdims=True),
                   axis=0, keepdims=True)
    sel = (flat == bidx).astype(jnp.float32)
    parts = []
    for j in range(9):
        mj = modelsT_ref[j] * sel
        parts.append(jnp.sum(jnp.sum(mj, axis=1, keepdims=True), axis=0,
                             keepdims=True))
    parts.append(m)
    parts.extend([jnp.zeros_like(m)] * 6)
    best_ref[...] = jnp.concatenate(parts, axis=1)


def _argmax_select(scores2d, modelsT):
    return pl.pallas_call(
        _argmax_kernel,
        out_shape=jax.ShapeDtypeStruct((1, 16), jnp.float32),
    )(scores2d, modelsT)


def _transfer_err(h, x1, y1, x2, y2):
    hb = [_bf(v) for v in h]
    xb = _bf(x1)
    yb = _bf(y1)
    q0 = (hb[0] * xb + hb[1] * yb) + hb[2]
    q1 = (hb[3] * xb + hb[4] * yb) + hb[5]
    z = (hb[6] * xb + hb[7] * yb) + hb[8]
    zs = jnp.where(jnp.abs(z) > 1e-8, z, 1e-8)
    px = q0 / zs
    py = q1 / zs
    return (px - x2) ** 2 + (py - y2) ** 2


def _kps32_rows(kps_ref):
    x1 = kps_ref[0:8, :]
    y1 = kps_ref[8:16, :]
    x2 = kps_ref[16:24, :]
    y2 = kps_ref[24:32, :]
    return x1, y1, x2, y2


def _inlmask_kernel(best_ref, kps_ref, inl_ref):
    mm = best_ref[...]
    h = [mm[:, j:j + 1] for j in range(9)]
    x1, y1, x2, y2 = _kps32_rows(kps_ref)
    err = _transfer_err(h, x1, y1, x2, y2)
    r_iota = lax.broadcasted_iota(jnp.int32, (PR, PC), 0)
    c_iota = lax.broadcasted_iota(jnp.int32, (PR, PC), 1)
    pidx = r_iota * PC + c_iota
    inl = (err <= INL_TH) & (pidx < N)
    inl_ref[...] = inl.astype(jnp.float32)


def _inlier_mask(best16, kps32):
    return pl.pallas_call(
        _inlmask_kernel,
        out_shape=jax.ShapeDtypeStruct((PR, PC), jnp.float32),
    )(best16, kps32)


def _fsum(x):
    return jnp.sum(jnp.sum(x, axis=1, keepdims=True), axis=0, keepdims=True)


def _dlt_full(x1, y1, x2, y2, w, mask):
    cnt = _fsum(mask)
    denom = jnp.where(cnt > 0.0, cnt, 1.0)

    def normN(xs, ys):
        mx = _fsum(xs * mask) / denom
        my = _fsum(ys * mask) / denom
        d = jnp.sqrt((xs - mx) ** 2 + (ys - my) ** 2)
        sc = _fsum(d * mask) / denom
        sc = SQRT2 / (sc + 1e-8)
        xn = (xs - mx) * sc
        yn = (ys - my) * sc
        T = [[sc, 0.0, -sc * mx], [0.0, sc, -sc * my], [0.0, 0.0, 1.0]]
        return xn, yn, T

    x1n, y1n, T1 = normN(x1, y1)
    x2n, y2n, T2 = normN(x2, y2)
    one = jnp.ones_like(x1n)

    def accumulate(x1n, y1n, x2n, y2n):
        ax, ay = _ax_ay(x1n, y1n, x2n, y2n, one)
        axL = [None if e is None else _bf(e) for e in ax]
        ayL = [None if e is None else _bf(e) for e in ay]
        axR = [None if e is None else _bf(e * w) for e in ax]
        ayR = [None if e is None else _bf(e * w) for e in ay]
        AtA = {}
        for j in range(9):
            for k in range(9):
                term = None
                if axL[j] is not None and axR[k] is not None:
                    term = axL[j] * axR[k]
                if ayL[j] is not None and ayR[k] is not None:
                    t2 = ayL[j] * ayR[k]
                    term = t2 if term is None else term + t2
                if term is None:
                    AtA[(j, k)] = jnp.zeros((1, 1), jnp.float32)
                else:
                    AtA[(j, k)] = _fsum(term)
        return AtA

    return _dlt_core(x1n, y1n, x2n, y2n, T1, T2, accumulate)


def _lo_kernel(kps_ref, w_ref, model_ref, inl_ref, score_ref):
    x1, y1, x2, y2 = _kps32_rows(kps_ref)
    w0 = w_ref[...]
    H = _dlt_full(x1, y1, x2, y2, w0, w0)
    for _ in range(5):
        h = [H[i][j] for i in range(3) for j in range(3)]
        err = _transfer_err(h, x1, y1, x2, y2)
        ws = w0 * jnp.exp(-err / 18.0)
        H = _dlt_full(x1, y1, x2, y2, ws, w0)
    h = [H[i][j] for i in range(3) for j in range(3)]
    err = _transfer_err(h, x1, y1, x2, y2)
    r_iota = lax.broadcasted_iota(jnp.int32, (PR, PC), 0)
    c_iota = lax.broadcasted_iota(jnp.int32, (PR, PC), 1)
    pidx = r_iota * PC + c_iota
    inl = (err <= INL_TH) & (pidx < N)
    inlf = inl.astype(jnp.float32)
    inl_ref[...] = inlf
    score_ref[...] = _fsum(inlf)
    model_ref[...] = jnp.concatenate(h + [jnp.zeros((1, 1), jnp.float32)] * 7,
                                     axis=1)


def _lo_iter(kps32, w):
    return pl.pallas_call(
        _lo_kernel,
        out_shape=(jax.ShapeDtypeStruct((1, 16), jnp.float32),
                   jax.ShapeDtypeStruct((PR, PC), jnp.float32),
                   jax.ShapeDtypeStruct((1, 1), jnp.float32)),
    )(kps32, w)


def kernel(kp1, kp2, weights=None):
    del weights
    f32 = jnp.float32
    kp1 = kp1.astype(f32)
    kp2 = kp2.astype(f32)

    base = jnp.stack([kp1[:, 0], kp1[:, 1], kp2[:, 0], kp2[:, 1]], axis=0)
    base = jnp.pad(base, ((0, 0), (0, NP - N)))
    kps_wide = jnp.pad(base, ((0, 4), (0, 0)))
    kps32 = base.reshape(4 * PR, PC)

    key = jax.random.key(42)
    sks = []
    for _ in range(4):
        key, sk = jax.random.split(key)
        sks.append(lax.bitcast_convert_type(jax.random.key_data(sk),
                                            jnp.int32))

    num_tc = N

    def lo_body(s):
        model16, inl0, sc, _act = s
        m_lo, inl_lo, s_lo = _lo_iter(kps32, inl0)
        s_lo = s_lo[0, 0]
        take = s_lo > sc
        return (jnp.where(take, m_lo, model16),
                jnp.where(take, inl_lo, inl0),
                jnp.where(take, s_lo, sc),
                take)

    def make_body(i):
        def body(state):
            bm16, binl, bscore, active = state
            coords = _sample(sks[i], kps_wide)
            coordsT = coords.T.reshape(16, 8, 128)
            modelsT, valid = _dlt4(coordsT)
            models_r = modelsT.reshape(9, B).T
            valid_r = valid.reshape(B, 1)
            scores = _verify_batch(models_r, kps_wide, valid_r)
            best16 = _argmax_select(scores.reshape(8, 128), modelsT)
            score = best16[0, 9]
            inl = _inlier_mask(best16, kps32)
            improved = score > bscore

            def lo_chain(args):
                model16, inl0, sc = args
                st = (model16, inl0, sc, jnp.asarray(True))
                for _ in range(5):
                    st = lax.cond(st[3], lo_body, lambda s: s, st)
                return st[:3]

            model_f, inl_f, score_f = lax.cond(
                improved, lo_chain, lambda a: a, (best16, inl, score))

            bm16n = jnp.where(improved, model_f, bm16)
            binln = jnp.where(improved, inl_f, binl)
            bsn = jnp.where(improved, score_f, bscore)
            n_inl = jnp.floor(bsn)
            ratio = n_inl / num_tc
            new_max_iter = jnp.where(
                n_inl == num_tc, 1.0,
                jnp.log(1.0 - 0.99) / jnp.log(1.0 - ratio ** 4))
            stop = improved & ((i + 1) * B >= jnp.floor(new_max_iter))
            return (bm16n, binln, bsn, active & ~stop)
        return body

    state = (jnp.zeros((1, 16), f32),
             jnp.zeros((PR, PC), f32),
             jnp.asarray(4.0, f32),
             jnp.asarray(True))
    for i in range(4):
        state = lax.cond(state[3], make_body(i), lambda s: s, state)

    bm16, binl, _bs, _a = state
    best_model = bm16[0, :9].reshape(3, 3)
    inliers = binl.reshape(NP)[:N].astype(bool)
    return best_model, inliers


# speedup vs baseline: 12.8221x; 1.0633x over previous
import jax
import jax.numpy as jnp
from jax import lax
from jax.experimental import pallas as pl
from jax.experimental.pallas import tpu as pltpu

N = 10000
NP = 10240
PR, PC = 8, 1280
B = 1024
HB = 8
INL_TH = 2.0
SQRT2 = 1.4142135623730951


def _bf(v):
    return lax.convert_element_type(
        lax.convert_element_type(v, jnp.bfloat16), jnp.float32)


def _rotl(x, r):
    return lax.shift_left(x, jnp.int32(r)) | lax.shift_right_logical(
        x, jnp.int32(32 - r))


def _threefry_bits(ks0, ks1, x1):
    ks2 = ks0 ^ ks1 ^ jnp.int32(0x1BD11BDA)
    ks = [ks0, ks1, ks2]
    rots = [[13, 15, 26, 6], [17, 29, 16, 24]]
    x0 = jnp.zeros_like(x1) + ks0
    x1 = x1 + ks1
    for i in range(5):
        for r in rots[i % 2]:
            x0 = x0 + x1
            x1 = _rotl(x1, r)
            x1 = x0 ^ x1
        x0 = x0 + ks[(i + 1) % 3]
        x1 = x1 + ks[(i + 2) % 3] + jnp.int32(i + 1)
    return x0 ^ x1


_CHUNK = 1024


def _sample_kernel(key_ref, kps_ref, coords_ref, v_scr):
    g = pl.program_id(0)
    h0 = g * HB
    ks0 = key_ref[0]
    ks1 = key_ref[1]

    def chunk_body(ci, carry):
        r_iota = lax.broadcasted_iota(jnp.int32, (HB, _CHUNK), 0)
        cc = lax.broadcasted_iota(jnp.int32, (HB, _CHUNK), 1) + ci * _CHUNK
        L = (h0 + r_iota) * N + cc
        bits = _threefry_bits(ks0, ks1, L)
        vv = lax.shift_right_logical(bits, jnp.int32(9))
        vv = jnp.where(cc < N, vv, -1)
        v_scr[ci] = vv
        return carry

    lax.fori_loop(0, NP // _CHUNK, chunk_body, 0)
    c_iota = lax.broadcasted_iota(jnp.int32, (HB, NP), 1)
    v = jnp.concatenate([v_scr[i] for i in range(NP // _CHUNK)], axis=1)
    coord_rows = [kps_ref[i:i + 1, :] for i in range(4)]
    out = [[], [], [], []]
    for _t in range(4):
        m = jnp.max(v, axis=1, keepdims=True)
        idx = jnp.min(jnp.where(v == m, c_iota, NP), axis=1, keepdims=True)
        sel = c_iota == idx
        for s in range(4):
            out[s].append(
                jnp.sum(jnp.where(sel, coord_rows[s], 0.0), axis=1,
                        keepdims=True))
        v = jnp.where(sel, -1, v)
    coords_ref[...] = jnp.concatenate(out[0] + out[1] + out[2] + out[3],
                                      axis=1)


def _sample(key2, kps_wide):
    return pl.pallas_call(
        _sample_kernel,
        grid=(B // HB,),
        in_specs=[
            pl.BlockSpec(memory_space=pltpu.SMEM),
            pl.BlockSpec((8, NP), lambda g: (0, 0)),
        ],
        out_specs=pl.BlockSpec((HB, 16), lambda g: (g, 0)),
        out_shape=jax.ShapeDtypeStruct((B, 16), jnp.float32),
        scratch_shapes=[pltpu.VMEM((NP // _CHUNK, HB, _CHUNK), jnp.int32)],
    )(key2, kps_wide)


def _inv3(T):
    a, b, c = T[0]
    d, e, f = T[1]
    g, h, i = T[2]
    A11 = e * i - f * h
    A12 = c * h - b * i
    A13 = b * f - c * e
    A21 = f * g - d * i
    A22 = a * i - c * g
    A23 = c * d - a * f
    A31 = d * h - e * g
    A32 = b * g - a * h
    A33 = a * e - b * d
    det = a * A11 + b * A21 + c * A31
    return [[A11 / det, A12 / det, A13 / det],
            [A21 / det, A22 / det, A23 / det],
            [A31 / det, A32 / det, A33 / det]]


def _bfv(v):
    if isinstance(v, (int, float)):
        return v
    return _bf(v)


def _mat3bf(Ma, Mb):
    out = []
    for i in range(3):
        row = []
        for j in range(3):
            p = [_bfv(Ma[i][k]) * _bfv(Mb[k][j]) for k in range(3)]
            row.append((p[0] + p[1]) + p[2])
        out.append(row)
    return out


def _solve8(AtA):
    def at(j, k):
        return AtA[(j, k)]

    M = [[at(j, k) + (1e-8 if j == k else 0.0) for k in range(8)]
         + [-at(j, 8)] for j in range(8)]
    for k in range(8):
        piv = M[k][k]
        for r in range(k + 1, 8):
            f = M[r][k] / piv
            for c in range(k + 1, 9):
                M[r][c] = M[r][c] - f * M[k][c]
    xs = [None] * 8
    for k in range(7, -1, -1):
        s = M[k][8]
        for c in range(k + 1, 8):
            s = s - M[k][c] * xs[c]
        xs[k] = s / M[k][k]
    return xs + [None]


def _dlt_core(x1n, y1n, x2n, y2n, T1, T2, accumulate):
    AtA = accumulate(x1n, y1n, x2n, y2n)
    xs = _solve8(AtA)
    one = jnp.ones_like(xs[0])
    Hn = [[xs[0], xs[1], xs[2]], [xs[3], xs[4], xs[5]], [xs[6], xs[7], one]]
    H = _mat3bf(_mat3bf(_inv3(T2), Hn), T1)
    z22 = H[2][2] + 1e-8
    return [[H[i][j] / z22 for j in range(3)] for i in range(3)]


def _ax_ay(X1, Y1, X2, Y2, one):
    ax = [None, None, None, -X1, -Y1, -one, Y2 * X1, Y2 * Y1, Y2]
    ay = [X1, Y1, one, None, None, None, (-X2) * X1, (-X2) * Y1, -X2]
    return ax, ay


def _dlt4_kernel(ct_ref, models_ref, valid_ref):
    x1 = [ct_ref[i] for i in range(4)]
    y1 = [ct_ref[4 + i] for i in range(4)]
    x2 = [ct_ref[8 + i] for i in range(4)]
    y2 = [ct_ref[12 + i] for i in range(4)]

    def norm4(xs, ys):
        mx = (((xs[0] + xs[1]) + xs[2]) + xs[3]) * 0.25
        my = (((ys[0] + ys[1]) + ys[2]) + ys[3]) * 0.25
        d = [jnp.sqrt((xs[i] - mx) ** 2 + (ys[i] - my) ** 2)
             for i in range(4)]
        sc = (((d[0] + d[1]) + d[2]) + d[3]) * 0.25
        sc = SQRT2 / (sc + 1e-8)
        xn = [(xs[i] - mx) * sc for i in range(4)]
        yn = [(ys[i] - my) * sc for i in range(4)]
        T = [[sc, 0.0, -sc * mx], [0.0, sc, -sc * my], [0.0, 0.0, 1.0]]
        return xn, yn, T

    x1n, y1n, T1 = norm4(x1, y1)
    x2n, y2n, T2 = norm4(x2, y2)
    one = jnp.ones_like(x1n[0])

    def accumulate(x1n, y1n, x2n, y2n):
        rows = []
        for i in range(4):
            ax, ay = _ax_ay(x1n[i], y1n[i], x2n[i], y2n[i], one)
            rows.append(ax)
        for i in range(4):
            ax, ay = _ax_ay(x1n[i], y1n[i], x2n[i], y2n[i], one)
            rows.append(ay)
        rows_bf = [[None if e is None else _bf(e) for e in row]
                   for row in rows]
        AtA = {}
        for row in rows_bf:
            for j in range(9):
                if row[j] is None:
                    continue
                for k in range(j, 9):
                    if row[k] is None:
                        continue
                    p = row[j] * row[k]
                    AtA[(j, k)] = AtA.get((j, k), 0.0) + p
        for j in range(9):
            for k in range(j, 9):
                AtA.setdefault((j, k), jnp.zeros_like(one))
                AtA[(k, j)] = AtA[(j, k)]
        return AtA

    H = _dlt_core(x1n, y1n, x2n, y2n, T1, T2, accumulate)

    for i in range(3):
        for j in range(3):
            models_ref[3 * i + j] = H[i][j]

    TRIPLETS = [(0, 1, 2), (0, 1, 3), (0, 2, 3), (1, 2, 3)]

    def det3h(xs, ys, t):
        xa, xb, xc = xs[t[0]], xs[t[1]], xs[t[2]]
        ya, yb, yc = ys[t[0]], ys[t[1]], ys[t[2]]
        return (xa * (yb - yc) - ya * (xb - xc)) + (xb * yc - yb * xc)

    mask = None
    for t in TRIPLETS:
        d1 = det3h(x1, y1, t)
        d2 = det3h(x2, y2, t)
        ok = (d1 * d2) > 0.0
        mask = ok if mask is None else (mask & ok)
    absd = [jnp.abs(H[0][0]), jnp.abs(H[1][1]), jnp.abs(H[2][2])]
    good = jnp.minimum(jnp.minimum(absd[0], absd[1]), absd[2]) > 1e-4
    valid_ref[...] = (mask & good).astype(jnp.float32)


def _dlt4(coordsT):
    return pl.pallas_call(
        _dlt4_kernel,
        out_shape=(jax.ShapeDtypeStruct((9, 8, 128), jnp.float32),
                   jax.ShapeDtypeStruct((8, 128), jnp.float32)),
    )(coordsT)


def _verify_kernel(models_ref, kps_ref, valid_ref, scores_ref):
    mm = models_ref[...]
    h = [mm[:, j:j + 1] for j in range(9)]
    x1 = kps_ref[0:1, :]
    y1 = kps_ref[1:2, :]
    x2 = kps_ref[2:3, :]
    y2 = kps_ref[3:4, :]
    c_iota = lax.broadcasted_iota(jnp.int32, (HB, NP), 1)
    err = _transfer_err(h, x1, y1, x2, y2)
    inl = (err <= INL_TH) & (c_iota < N)
    score = jnp.sum(inl.astype(jnp.float32), axis=1, keepdims=True)
    scores_ref[...] = jnp.where(valid_ref[...] > 0.0, score, -1.0)


def _verify_batch(models_r, kps_wide, valid_r):
    return pl.pallas_call(
        _verify_kernel,
        grid=(B // HB,),
        in_specs=[
            pl.BlockSpec((HB, 9), lambda g: (g, 0)),
            pl.BlockSpec((8, NP), lambda g: (0, 0)),
            pl.BlockSpec((HB, 1), lambda g: (g, 0)),
        ],
        out_specs=pl.BlockSpec((HB, 1), lambda g: (g, 0)),
        out_shape=jax.ShapeDtypeStruct((B, 1), jnp.float32),
    )(models_r, kps_wide, valid_r)


def _argmax_kernel(scores_ref, modelsT_ref, best_ref):
    s = scores_ref[...]
    m = jnp.max(jnp.max(s, axis=1, keepdims=True), axis=0, keepdims=True)
    r_iota = lax.broadcasted_iota(jnp.int32, (8, 128), 0)
    c_iota = lax.broadcasted_iota(jnp.int32, (8, 128), 1)
    flat = r_iota * 128 + c_iota
    eq = s == m
    bidx = jnp.min(jnp.min(jnp.where(eq, flat, B), axis=1, keepdims=True),
                   axis=0, keepdims=True)
    sel = (flat == bidx).astype(jnp.float32)
    parts = []
    for j in range(9):
        mj = modelsT_ref[j] * sel
        parts.append(jnp.sum(jnp.sum(mj, axis=1, keepdims=True), axis=0,
                             keepdims=True))
    parts.append(m)
    parts.extend([jnp.zeros_like(m)] * 6)
    best_ref[...] = jnp.concatenate(parts, axis=1)


def _argmax_select(scores2d, modelsT):
    return pl.pallas_call(
        _argmax_kernel,
        out_shape=jax.ShapeDtypeStruct((1, 16), jnp.float32),
    )(scores2d, modelsT)


def _transfer_err(h, x1, y1, x2, y2):
    hb = [_bf(v) for v in h]
    xb = _bf(x1)
    yb = _bf(y1)
    q0 = (hb[0] * xb + hb[1] * yb) + hb[2]
    q1 = (hb[3] * xb + hb[4] * yb) + hb[5]
    z = (hb[6] * xb + hb[7] * yb) + hb[8]
    zs = jnp.where(jnp.abs(z) > 1e-8, z, 1e-8)
    px = q0 / zs
    py = q1 / zs
    return (px - x2) ** 2 + (py - y2) ** 2


def _kps32_rows(kps_ref):
    x1 = kps_ref[0:8, :]
    y1 = kps_ref[8:16, :]
    x2 = kps_ref[16:24, :]
    y2 = kps_ref[24:32, :]
    return x1, y1, x2, y2


def _inlmask_kernel(best_ref, kps_ref, inl_ref):
    mm = best_ref[...]
    h = [mm[:, j:j + 1] for j in range(9)]
    x1, y1, x2, y2 = _kps32_rows(kps_ref)
    err = _transfer_err(h, x1, y1, x2, y2)
    r_iota = lax.broadcasted_iota(jnp.int32, (PR, PC), 0)
    c_iota = lax.broadcasted_iota(jnp.int32, (PR, PC), 1)
    pidx = r_iota * PC + c_iota
    inl = (err <= INL_TH) & (pidx < N)
    inl_ref[...] = inl.astype(jnp.float32)


def _inlier_mask(best16, kps32):
    return pl.pallas_call(
        _inlmask_kernel,
        out_shape=jax.ShapeDtypeStruct((PR, PC), jnp.float32),
    )(best16, kps32)


def _fsum(x):
    return jnp.sum(jnp.sum(x, axis=1, keepdims=True), axis=0, keepdims=True)


def _dlt_full(x1, y1, x2, y2, w, mask):
    cnt = _fsum(mask)
    denom = jnp.where(cnt > 0.0, cnt, 1.0)

    def normN(xs, ys):
        mx = _fsum(xs * mask) / denom
        my = _fsum(ys * mask) / denom
        d = jnp.sqrt((xs - mx) ** 2 + (ys - my) ** 2)
        sc = _fsum(d * mask) / denom
        sc = SQRT2 / (sc + 1e-8)
        xn = (xs - mx) * sc
        yn = (ys - my) * sc
        T = [[sc, 0.0, -sc * mx], [0.0, sc, -sc * my], [0.0, 0.0, 1.0]]
        return xn, yn, T

    x1n, y1n, T1 = normN(x1, y1)
    x2n, y2n, T2 = normN(x2, y2)
    one = jnp.ones_like(x1n)

    def accumulate(x1n, y1n, x2n, y2n):
        ax, ay = _ax_ay(x1n, y1n, x2n, y2n, one)
        axL = [None if e is None else _bf(e) for e in ax]
        ayL = [None if e is None else _bf(e) for e in ay]
        axR = [None if e is None else _bf(e * w) for e in ax]
        ayR = [None if e is None else _bf(e * w) for e in ay]
        AtA = {}
        for j in range(9):
            for k in range(9):
                term = None
                if axL[j] is not None and axR[k] is not None:
                    term = axL[j] * axR[k]
                if ayL[j] is not None and ayR[k] is not None:
                    t2 = ayL[j] * ayR[k]
                    term = t2 if term is None else term + t2
                if term is None:
                    AtA[(j, k)] = jnp.zeros((1, 1), jnp.float32)
                else:
                    AtA[(j, k)] = _fsum(term)
        return AtA

    return _dlt_core(x1n, y1n, x2n, y2n, T1, T2, accumulate)


def _lo_kernel(kps_ref, w_ref, model_ref, inl_ref, score_ref):
    x1, y1, x2, y2 = _kps32_rows(kps_ref)
    w0 = w_ref[...]
    H = _dlt_full(x1, y1, x2, y2, w0, w0)
    for _ in range(5):
        h = [H[i][j] for i in range(3) for j in range(3)]
        err = _transfer_err(h, x1, y1, x2, y2)
        ws = w0 * jnp.exp(-err / 18.0)
        H = _dlt_full(x1, y1, x2, y2, ws, w0)
    h = [H[i][j] for i in range(3) for j in range(3)]
    err = _transfer_err(h, x1, y1, x2, y2)
    r_iota = lax.broadcasted_iota(jnp.int32, (PR, PC), 0)
    c_iota = lax.broadcasted_iota(jnp.int32, (PR, PC), 1)
    pidx = r_iota * PC + c_iota
    inl = (err <= INL_TH) & (pidx < N)
    inlf = inl.astype(jnp.float32)
    inl_ref[...] = inlf
    score_ref[...] = _fsum(inlf)
    model_ref[...] = jnp.concatenate(h + [jnp.zeros((1, 1), jnp.float32)] * 7,
                                     axis=1)


def _lo_iter(kps32, w):
    return pl.pallas_call(
        _lo_kernel,
        out_shape=(jax.ShapeDtypeStruct((1, 16), jnp.float32),
                   jax.ShapeDtypeStruct((PR, PC), jnp.float32),
                   jax.ShapeDtypeStruct((1, 1), jnp.float32)),
    )(kps32, w)


def kernel(kp1, kp2, weights=None):
    del weights
    f32 = jnp.float32
    kp1 = kp1.astype(f32)
    kp2 = kp2.astype(f32)

    base = jnp.stack([kp1[:, 0], kp1[:, 1], kp2[:, 0], kp2[:, 1]], axis=0)
    base = jnp.pad(base, ((0, 0), (0, NP - N)))
    kps_wide = jnp.pad(base, ((0, 4), (0, 0)))
    kps32 = base.reshape(4 * PR, PC)

    key = jax.random.key(42)
    sks = []
    for _ in range(4):
        key, sk = jax.random.split(key)
        sks.append(lax.bitcast_convert_type(jax.random.key_data(sk),
                                            jnp.int32))

    num_tc = N

    def lo_body(s):
        model16, inl0, sc, _act = s
        m_lo, inl_lo, s_lo = _lo_iter(kps32, inl0)
        s_lo = s_lo[0, 0]
        take = s_lo > sc
        return (jnp.where(take, m_lo, model16),
                jnp.where(take, inl_lo, inl0),
                jnp.where(take, s_lo, sc),
                take)

    def make_body(i):
        def body(state):
            bm16, binl, bscore, active = state
            coords = _sample(sks[i], kps_wide)
            coordsT = coords.T.reshape(16, 8, 128)
            modelsT, valid = _dlt4(coordsT)
            models_r = modelsT.reshape(9, B).T
            valid_r = valid.reshape(B, 1)
            scores = _verify_batch(models_r, kps_wide, valid_r)
            best16 = _argmax_select(scores.reshape(8, 128), modelsT)
            score = best16[0, 9]
            inl = _inlier_mask(best16, kps32)
            improved = score > bscore

            def lo_chain(args):
                model16, inl0, sc = args
                st = (model16, inl0, sc, jnp.asarray(True))
                for _ in range(5):
                    st = lax.cond(st[3], lo_body, lambda s: s, st)
                return st[:3]

            model_f, inl_f, score_f = lax.cond(
                improved, lo_chain, lambda a: a, (best16, inl, score))

            bm16n = jnp.where(improved, model_f, bm16)
            binln = jnp.where(improved, inl_f, binl)
            bsn = jnp.where(improved, score_f, bscore)
            n_inl = jnp.floor(bsn)
            ratio = n_inl / num_tc
            new_max_iter = jnp.where(
                n_inl == num_tc, 1.0,
                jnp.log(1.0 - 0.99) / jnp.log(1.0 - ratio ** 4))
            stop = improved & ((i + 1) * B >= jnp.floor(new_max_iter))
            return (bm16n, binln, bsn, active & ~stop)
        return body

    state = (jnp.zeros((1, 16), f32),
             jnp.zeros((PR, PC), f32),
             jnp.asarray(4.0, f32),
             jnp.asarray(True))
    for i in range(4):
        state = lax.cond(state[3], make_body(i), lambda s: s, state)

    bm16, binl, _bs, _a = state
    best_model = bm16[0, :9].reshape(3, 3)
    inliers = binl.reshape(NP)[:N].astype(bool)
    return best_model, inliers


# speedup vs baseline: 13.1211x; 1.0233x over previous
import jax
import jax.numpy as jnp
from jax import lax
from jax.experimental import pallas as pl
from jax.experimental.pallas import tpu as pltpu

N = 10000
NP = 10240
PR, PC = 8, 1280
B = 1024
HB = 8
INL_TH = 2.0
SQRT2 = 1.4142135623730951


def _bf(v):
    return lax.convert_element_type(
        lax.convert_element_type(v, jnp.bfloat16), jnp.float32)


def _rotl(x, r):
    return lax.shift_left(x, jnp.int32(r)) | lax.shift_right_logical(
        x, jnp.int32(32 - r))


def _threefry_bits(ks0, ks1, x1):
    ks2 = ks0 ^ ks1 ^ jnp.int32(0x1BD11BDA)
    ks = [ks0, ks1, ks2]
    rots = [[13, 15, 26, 6], [17, 29, 16, 24]]
    x0 = jnp.zeros_like(x1) + ks0
    x1 = x1 + ks1
    for i in range(5):
        for r in rots[i % 2]:
            x0 = x0 + x1
            x1 = _rotl(x1, r)
            x1 = x0 ^ x1
        x0 = x0 + ks[(i + 1) % 3]
        x1 = x1 + ks[(i + 2) % 3] + jnp.int32(i + 1)
    return x0 ^ x1


_CHUNK = 1024


def _sample_kernel(key_ref, kps_ref, coords_ref, v_scr):
    g = pl.program_id(0)
    h0 = g * HB
    ks0 = key_ref[0]
    ks1 = key_ref[1]

    def chunk_body(ci, carry):
        r_iota = lax.broadcasted_iota(jnp.int32, (HB, _CHUNK), 0)
        cc = lax.broadcasted_iota(jnp.int32, (HB, _CHUNK), 1) + ci * _CHUNK
        L = (h0 + r_iota) * N + cc
        bits = _threefry_bits(ks0, ks1, L)
        vv = lax.shift_right_logical(bits, jnp.int32(9))
        vv = jnp.where(cc < N, vv, -1)
        v_scr[ci] = vv
        return carry

    lax.fori_loop(0, NP // _CHUNK, chunk_body, 0)
    c_iota = lax.broadcasted_iota(jnp.int32, (HB, NP), 1)
    v = jnp.concatenate([v_scr[i] for i in range(NP // _CHUNK)], axis=1)
    coord_rows = [kps_ref[i:i + 1, :] for i in range(4)]
    out = [[], [], [], []]
    for _t in range(4):
        m = jnp.max(v, axis=1, keepdims=True)
        idx = jnp.min(jnp.where(v == m, c_iota, NP), axis=1, keepdims=True)
        sel = c_iota == idx
        for s in range(4):
            out[s].append(
                jnp.sum(jnp.where(sel, coord_rows[s], 0.0), axis=1,
                        keepdims=True))
        v = jnp.where(sel, -1, v)
    coords_ref[...] = jnp.concatenate(out[0] + out[1] + out[2] + out[3],
                                      axis=1)


def _sample(key2, kps_wide):
    return pl.pallas_call(
        _sample_kernel,
        grid=(B // HB,),
        in_specs=[
            pl.BlockSpec(memory_space=pltpu.SMEM),
            pl.BlockSpec((8, NP), lambda g: (0, 0)),
        ],
        out_specs=pl.BlockSpec((HB, 16), lambda g: (g, 0)),
        out_shape=jax.ShapeDtypeStruct((B, 16), jnp.float32),
        scratch_shapes=[pltpu.VMEM((NP // _CHUNK, HB, _CHUNK), jnp.int32)],
    )(key2, kps_wide)


def _inv3(T):
    a, b, c = T[0]
    d, e, f = T[1]
    g, h, i = T[2]
    A11 = e * i - f * h
    A12 = c * h - b * i
    A13 = b * f - c * e
    A21 = f * g - d * i
    A22 = a * i - c * g
    A23 = c * d - a * f
    A31 = d * h - e * g
    A32 = b * g - a * h
    A33 = a * e - b * d
    det = a * A11 + b * A21 + c * A31
    return [[A11 / det, A12 / det, A13 / det],
            [A21 / det, A22 / det, A23 / det],
            [A31 / det, A32 / det, A33 / det]]


def _bfv(v):
    if isinstance(v, (int, float)):
        return v
    return _bf(v)


def _mat3bf(Ma, Mb):
    out = []
    for i in range(3):
        row = []
        for j in range(3):
            p = [_bfv(Ma[i][k]) * _bfv(Mb[k][j]) for k in range(3)]
            row.append((p[0] + p[1]) + p[2])
        out.append(row)
    return out


def _solve8(AtA):
    def at(j, k):
        return AtA[(j, k)]

    M = [[at(j, k) + (1e-8 if j == k else 0.0) for k in range(8)]
         + [-at(j, 8)] for j in range(8)]
    for k in range(8):
        piv = M[k][k]
        for r in range(k + 1, 8):
            f = M[r][k] / piv
            for c in range(k + 1, 9):
                M[r][c] = M[r][c] - f * M[k][c]
    xs = [None] * 8
    for k in range(7, -1, -1):
        s = M[k][8]
        for c in range(k + 1, 8):
            s = s - M[k][c] * xs[c]
        xs[k] = s / M[k][k]
    return xs + [None]


def _dlt_core(x1n, y1n, x2n, y2n, T1, T2, accumulate):
    AtA = accumulate(x1n, y1n, x2n, y2n)
    xs = _solve8(AtA)
    one = jnp.ones_like(xs[0])
    Hn = [[xs[0], xs[1], xs[2]], [xs[3], xs[4], xs[5]], [xs[6], xs[7], one]]
    H = _mat3bf(_mat3bf(_inv3(T2), Hn), T1)
    z22 = H[2][2] + 1e-8
    return [[H[i][j] / z22 for j in range(3)] for i in range(3)]


def _ax_ay(X1, Y1, X2, Y2, one):
    ax = [None, None, None, -X1, -Y1, -one, Y2 * X1, Y2 * Y1, Y2]
    ay = [X1, Y1, one, None, None, None, (-X2) * X1, (-X2) * Y1, -X2]
    return ax, ay


def _dlt4_kernel(ct_ref, models_ref, valid_ref):
    x1 = [ct_ref[i] for i in range(4)]
    y1 = [ct_ref[4 + i] for i in range(4)]
    x2 = [ct_ref[8 + i] for i in range(4)]
    y2 = [ct_ref[12 + i] for i in range(4)]

    def norm4(xs, ys):
        mx = (((xs[0] + xs[1]) + xs[2]) + xs[3]) * 0.25
        my = (((ys[0] + ys[1]) + ys[2]) + ys[3]) * 0.25
        d = [jnp.sqrt((xs[i] - mx) ** 2 + (ys[i] - my) ** 2)
             for i in range(4)]
        sc = (((d[0] + d[1]) + d[2]) + d[3]) * 0.25
        sc = SQRT2 / (sc + 1e-8)
        xn = [(xs[i] - mx) * sc for i in range(4)]
        yn = [(ys[i] - my) * sc for i in range(4)]
        T = [[sc, 0.0, -sc * mx], [0.0, sc, -sc * my], [0.0, 0.0, 1.0]]
        return xn, yn, T

    x1n, y1n, T1 = norm4(x1, y1)
    x2n, y2n, T2 = norm4(x2, y2)
    one = jnp.ones_like(x1n[0])

    def accumulate(x1n, y1n, x2n, y2n):
        rows = []
        for i in range(4):
            ax, ay = _ax_ay(x1n[i], y1n[i], x2n[i], y2n[i], one)
            rows.append(ax)
        for i in range(4):
            ax, ay = _ax_ay(x1n[i], y1n[i], x2n[i], y2n[i], one)
            rows.append(ay)
        rows_bf = [[None if e is None else _bf(e) for e in row]
                   for row in rows]
        AtA = {}
        for row in rows_bf:
            for j in range(9):
                if row[j] is None:
                    continue
                for k in range(j, 9):
                    if row[k] is None:
                        continue
                    p = row[j] * row[k]
                    AtA[(j, k)] = AtA.get((j, k), 0.0) + p
        for j in range(9):
            for k in range(j, 9):
                AtA.setdefault((j, k), jnp.zeros_like(one))
                AtA[(k, j)] = AtA[(j, k)]
        return AtA

    H = _dlt_core(x1n, y1n, x2n, y2n, T1, T2, accumulate)

    for i in range(3):
        for j in range(3):
            models_ref[3 * i + j] = H[i][j]

    TRIPLETS = [(0, 1, 2), (0, 1, 3), (0, 2, 3), (1, 2, 3)]

    def det3h(xs, ys, t):
        xa, xb, xc = xs[t[0]], xs[t[1]], xs[t[2]]
        ya, yb, yc = ys[t[0]], ys[t[1]], ys[t[2]]
        return (xa * (yb - yc) - ya * (xb - xc)) + (xb * yc - yb * xc)

    mask = None
    for t in TRIPLETS:
        d1 = det3h(x1, y1, t)
        d2 = det3h(x2, y2, t)
        ok = (d1 * d2) > 0.0
        mask = ok if mask is None else (mask & ok)
    absd = [jnp.abs(H[0][0]), jnp.abs(H[1][1]), jnp.abs(H[2][2])]
    good = jnp.minimum(jnp.minimum(absd[0], absd[1]), absd[2]) > 1e-4
    valid_ref[...] = (mask & good).astype(jnp.float32)


def _dlt4(coordsT):
    return pl.pallas_call(
        _dlt4_kernel,
        out_shape=(jax.ShapeDtypeStruct((9, 8, 128), jnp.float32),
                   jax.ShapeDtypeStruct((8, 128), jnp.float32)),
    )(coordsT)


def _verify_kernel(models_ref, kps_ref, valid_ref, scores_ref):
    mm = models_ref[...]
    h = [mm[:, j:j + 1] for j in range(9)]
    x1 = kps_ref[0:1, :]
    y1 = kps_ref[1:2, :]
    x2 = kps_ref[2:3, :]
    y2 = kps_ref[3:4, :]
    c_iota = lax.broadcasted_iota(jnp.int32, (HB, NP), 1)
    err = _transfer_err(h, x1, y1, x2, y2)
    inl = (err <= INL_TH) & (c_iota < N)
    score = jnp.sum(inl.astype(jnp.float32), axis=1, keepdims=True)
    scores_ref[...] = jnp.where(valid_ref[...] > 0.0, score, -1.0)


def _verify_batch(models_r, kps_wide, valid_r):
    return pl.pallas_call(
        _verify_kernel,
        grid=(B // HB,),
        in_specs=[
            pl.BlockSpec((HB, 9), lambda g: (g, 0)),
            pl.BlockSpec((8, NP), lambda g: (0, 0)),
            pl.BlockSpec((HB, 1), lambda g: (g, 0)),
        ],
        out_specs=pl.BlockSpec((HB, 1), lambda g: (g, 0)),
        out_shape=jax.ShapeDtypeStruct((B, 1), jnp.float32),
    )(models_r, kps_wide, valid_r)


def _argmax_kernel(scores_ref, modelsT_ref, best_ref):
    s = scores_ref[...]
    m = jnp.max(jnp.max(s, axis=1, keepdims=True), axis=0, keepdims=True)
    r_iota = lax.broadcasted_iota(jnp.int32, (8, 128), 0)
    c_iota = lax.broadcasted_iota(jnp.int32, (8, 128), 1)
    flat = r_iota * 128 + c_iota
    eq = s == m
    bidx = jnp.min(jnp.min(jnp.where(eq, flat, B), axis=1, keepdims=True),
                   axis=0, keepdims=True)
    sel = (flat == bidx).astype(jnp.float32)
    parts = []
    for j in range(9):
        mj = modelsT_ref[j] * sel
        parts.append(jnp.sum(jnp.sum(mj, axis=1, keepdims=True), axis=0,
                             keepdims=True))
    parts.append(m)
    parts.extend([jnp.zeros_like(m)] * 6)
    best_ref[...] = jnp.concatenate(parts, axis=1)


def _argmax_select(scores2d, modelsT):
    return pl.pallas_call(
        _argmax_kernel,
        out_shape=jax.ShapeDtypeStruct((1, 16), jnp.float32),
    )(scores2d, modelsT)


def _transfer_err(h, x1, y1, x2, y2):
    hb = [_bf(v) for v in h]
    xb = _bf(x1)
    yb = _bf(y1)
    q0 = (hb[0] * xb + hb[1] * yb) + hb[2]
    q1 = (hb[3] * xb + hb[4] * yb) + hb[5]
    z = (hb[6] * xb + hb[7] * yb) + hb[8]
    zs = jnp.where(jnp.abs(z) > 1e-8, z, 1e-8)
    px = q0 / zs
    py = q1 / zs
    return (px - x2) ** 2 + (py - y2) ** 2


def _kps32_rows(kps_ref):
    x1 = kps_ref[0:8, :]
    y1 = kps_ref[8:16, :]
    x2 = kps_ref[16:24, :]
    y2 = kps_ref[24:32, :]
    return x1, y1, x2, y2


def _inlmask_kernel(best_ref, kps_ref, inl_ref):
    mm = best_ref[...]
    h = [mm[:, j:j + 1] for j in range(9)]
    x1, y1, x2, y2 = _kps32_rows(kps_ref)
    err = _transfer_err(h, x1, y1, x2, y2)
    r_iota = lax.broadcasted_iota(jnp.int32, (PR, PC), 0)
    c_iota = lax.broadcasted_iota(jnp.int32, (PR, PC), 1)
    pidx = r_iota * PC + c_iota
    inl = (err <= INL_TH) & (pidx < N)
    inl_ref[...] = inl.astype(jnp.float32)


def _inlier_mask(best16, kps32):
    return pl.pallas_call(
        _inlmask_kernel,
        out_shape=jax.ShapeDtypeStruct((PR, PC), jnp.float32),
    )(best16, kps32)


def _fsum(x):
    return jnp.sum(jnp.sum(x, axis=1, keepdims=True), axis=0, keepdims=True)


def _dlt_full(x1, y1, x2, y2, w, mask):
    cnt = _fsum(mask)
    denom = jnp.where(cnt > 0.0, cnt, 1.0)

    def normN(xs, ys):
        mx = _fsum(xs * mask) / denom
        my = _fsum(ys * mask) / denom
        d = jnp.sqrt((xs - mx) ** 2 + (ys - my) ** 2)
        sc = _fsum(d * mask) / denom
        sc = SQRT2 / (sc + 1e-8)
        xn = (xs - mx) * sc
        yn = (ys - my) * sc
        T = [[sc, 0.0, -sc * mx], [0.0, sc, -sc * my], [0.0, 0.0, 1.0]]
        return xn, yn, T

    x1n, y1n, T1 = normN(x1, y1)
    x2n, y2n, T2 = normN(x2, y2)
    one = jnp.ones_like(x1n)

    def accumulate(x1n, y1n, x2n, y2n):
        ax, ay = _ax_ay(x1n, y1n, x2n, y2n, one)
        axL = [None if e is None else _bf(e) for e in ax]
        ayL = [None if e is None else _bf(e) for e in ay]
        axR = [None if e is None else _bf(e * w) for e in ax]
        ayR = [None if e is None else _bf(e * w) for e in ay]
        AtA = {}
        for j in range(9):
            for k in range(9):
                term = None
                if axL[j] is not None and axR[k] is not None:
                    term = axL[j] * axR[k]
                if ayL[j] is not None and ayR[k] is not None:
                    t2 = ayL[j] * ayR[k]
                    term = t2 if term is None else term + t2
                if term is None:
                    AtA[(j, k)] = jnp.zeros((1, 1), jnp.float32)
                else:
                    AtA[(j, k)] = _fsum(term)
        return AtA

    return _dlt_core(x1n, y1n, x2n, y2n, T1, T2, accumulate)


def _lo_kernel(kps_ref, w_ref, model_ref, inl_ref, score_ref):
    x1, y1, x2, y2 = _kps32_rows(kps_ref)
    w0 = w_ref[...]
    H = _dlt_full(x1, y1, x2, y2, w0, w0)
    for _ in range(5):
        h = [H[i][j] for i in range(3) for j in range(3)]
        err = _transfer_err(h, x1, y1, x2, y2)
        ws = w0 * jnp.exp(-err / 18.0)
        H = _dlt_full(x1, y1, x2, y2, ws, w0)
    h = [H[i][j] for i in range(3) for j in range(3)]
    err = _transfer_err(h, x1, y1, x2, y2)
    r_iota = lax.broadcasted_iota(jnp.int32, (PR, PC), 0)
    c_iota = lax.broadcasted_iota(jnp.int32, (PR, PC), 1)
    pidx = r_iota * PC + c_iota
    inl = (err <= INL_TH) & (pidx < N)
    inlf = inl.astype(jnp.float32)
    inl_ref[...] = inlf
    score_ref[...] = _fsum(inlf)
    model_ref[...] = jnp.concatenate(h + [jnp.zeros((1, 1), jnp.float32)] * 7,
                                     axis=1)


def _lo_iter(kps32, w):
    return pl.pallas_call(
        _lo_kernel,
        out_shape=(jax.ShapeDtypeStruct((1, 16), jnp.float32),
                   jax.ShapeDtypeStruct((PR, PC), jnp.float32),
                   jax.ShapeDtypeStruct((1, 1), jnp.float32)),
    )(kps32, w)


def kernel(kp1, kp2, weights=None):
    del weights
    f32 = jnp.float32
    kp1 = kp1.astype(f32)
    kp2 = kp2.astype(f32)

    base = jnp.stack([kp1[:, 0], kp1[:, 1], kp2[:, 0], kp2[:, 1]], axis=0)
    base = jnp.pad(base, ((0, 0), (0, NP - N)))
    kps_wide = jnp.pad(base, ((0, 4), (0, 0)))
    kps32 = base.reshape(4 * PR, PC)

    key = jax.random.key(42)
    sks = []
    for _ in range(4):
        key, sk = jax.random.split(key)
        sks.append(lax.bitcast_convert_type(jax.random.key_data(sk),
                                            jnp.int32))
    sks_arr = jnp.stack(sks)

    num_tc = N

    def lo_body(s):
        model16, inl0, sc, _act = s
        m_lo, inl_lo, s_lo = _lo_iter(kps32, inl0)
        s_lo = s_lo[0, 0]
        take = s_lo > sc
        return (jnp.where(take, m_lo, model16),
                jnp.where(take, inl_lo, inl0),
                jnp.where(take, s_lo, sc),
                take)

    def while_cond(state):
        i, _bm, _bi, _bs, active = state
        return active & (i < 4)

    def while_body(state):
        i, bm16, binl, bscore, active = state
        sk = lax.dynamic_index_in_dim(sks_arr, i, axis=0, keepdims=False)
        coords = _sample(sk, kps_wide)
        coordsT = coords.T.reshape(16, 8, 128)
        modelsT, valid = _dlt4(coordsT)
        models_r = modelsT.reshape(9, B).T
        valid_r = valid.reshape(B, 1)
        scores = _verify_batch(models_r, kps_wide, valid_r)
        best16 = _argmax_select(scores.reshape(8, 128), modelsT)
        score = best16[0, 9]
        inl = _inlier_mask(best16, kps32)
        improved = score > bscore

        def lo_chain(args):
            model16, inl0, sc = args
            st = (model16, inl0, sc, jnp.asarray(True))
            for _ in range(5):
                st = lax.cond(st[3], lo_body, lambda s: s, st)
            return st[:3]

        model_f, inl_f, score_f = lax.cond(
            improved, lo_chain, lambda a: a, (best16, inl, score))

        bm16n = jnp.where(improved, model_f, bm16)
        binln = jnp.where(improved, inl_f, binl)
        bsn = jnp.where(improved, score_f, bscore)
        n_inl = jnp.floor(bsn)
        ratio = n_inl / num_tc
        new_max_iter = jnp.where(
            n_inl == num_tc, 1.0,
            jnp.log(1.0 - 0.99) / jnp.log(1.0 - ratio ** 4))
        stop = improved & ((i + 1) * B >= jnp.floor(new_max_iter))
        return (i + 1, bm16n, binln, bsn, active & ~stop)

    state = (jnp.asarray(0, jnp.int32),
             jnp.zeros((1, 16), f32),
             jnp.zeros((PR, PC), f32),
             jnp.asarray(4.0, f32),
             jnp.asarray(True))
    state = lax.while_loop(while_cond, while_body, state)

    _i, bm16, binl, _bs, _a = state
    best_model = bm16[0, :9].reshape(3, 3)
    inliers = binl.reshape(NP)[:N].astype(bool)
    return best_model, inliers
